# Initial kernel scaffold; baseline (speedup 1.0000x reference)
import jax
import jax.numpy as jnp
from jax import lax
from jax.experimental import pallas as pl
from jax.experimental.pallas import tpu as pltpu
from jax.experimental.pallas import tpu_sc as plsc

N = 10000
E = 320000
D = 128
H = 64
O = 64
K = 10
ALPHA = 0.1

NC = 2
NS = 16
NW = NC * NS
CHUNK = 128
NCH = 79
E_PAD = NW * NCH * CHUNK
N_PAD = 10240
ROWS_PER_TILE = N_PAD // NS


def _sc_prop_body(t_hbm, src_hbm, dst_hbm, zeros_hbm, out_hbm,
                  idx_s, idx_d, rows, agg):
    cid = lax.axis_index("c")
    sid = lax.axis_index("s")
    wid = cid * NS + sid

    pltpu.sync_copy(zeros_hbm.at[pl.ds(sid * ROWS_PER_TILE, ROWS_PER_TILE)],
                    agg.at[pl.ds(sid * ROWS_PER_TILE, ROWS_PER_TILE)])
    plsc.subcore_barrier()

    @pl.loop(0, NCH)
    def _(j):
        off = (wid * NCH + j) * CHUNK
        pltpu.sync_copy(src_hbm.at[pl.ds(off, CHUNK)], idx_s)
        pltpu.sync_copy(dst_hbm.at[pl.ds(off, CHUNK)], idx_d)
        pltpu.sync_copy(t_hbm.at[idx_s], rows)
        pltpu.sync_copy(rows, agg.at[idx_d], add=True)

    plsc.subcore_barrier()
    pltpu.sync_copy(agg.at[pl.ds(sid * ROWS_PER_TILE, ROWS_PER_TILE)],
                    out_hbm.at[cid].at[pl.ds(sid * ROWS_PER_TILE, ROWS_PER_TILE)])


@jax.jit
def _sc_propagate(t, src, dst, zeros):
    mesh = plsc.VectorSubcoreMesh(core_axis_name="c", subcore_axis_name="s")
    kern = pl.kernel(
        _sc_prop_body,
        out_type=jax.ShapeDtypeStruct((NC, N_PAD, O), jnp.float32),
        mesh=mesh,
        scratch_types=[
            pltpu.VMEM((CHUNK,), jnp.int32),
            pltpu.VMEM((CHUNK,), jnp.int32),
            pltpu.VMEM((CHUNK, O), jnp.float32),
            pltpu.VMEM_SHARED((N_PAD, O), jnp.float32),
        ],
    )
    return kern(t, src, dst, zeros)


_BLK = 1024
_GRID = N_PAD // _BLK


def _mlp_body(x_ref, w1_ref, b1_ref, w2_ref, b2_ref, h_ref):
    z = jnp.dot(x_ref[...], w1_ref[...], preferred_element_type=jnp.float32)
    z = jnp.maximum(z + b1_ref[...], 0.0)
    h_ref[...] = (jnp.dot(z, w2_ref[...], preferred_element_type=jnp.float32)
                  + b2_ref[...])


@jax.jit
def _mlp(x_pad, W1, b1, W2, b2):
    return pl.pallas_call(
        _mlp_body,
        grid=(_GRID,),
        in_specs=[
            pl.BlockSpec((_BLK, D), lambda i: (i, 0)),
            pl.BlockSpec((D, H), lambda i: (0, 0)),
            pl.BlockSpec((1, H), lambda i: (0, 0)),
            pl.BlockSpec((H, O), lambda i: (0, 0)),
            pl.BlockSpec((1, O), lambda i: (0, 0)),
        ],
        out_specs=pl.BlockSpec((_BLK, O), lambda i: (i, 0)),
        out_shape=jax.ShapeDtypeStruct((N_PAD, O), jnp.float32),
    )(x_pad, W1, b1, W2, b2)


def _update_body(a_ref, s_ref, p0_ref, p1_ref, t_ref, o_ref):
    o_ref[...] = a_ref[...] + s_ref[...] * (p0_ref[...] + p1_ref[...] + t_ref[...])


@jax.jit
def _update(a, s_col, p0, p1, t):
    return pl.pallas_call(
        _update_body,
        grid=(_GRID,),
        in_specs=[
            pl.BlockSpec((_BLK, O), lambda i: (i, 0)),
            pl.BlockSpec((_BLK, 1), lambda i: (i, 0)),
            pl.BlockSpec((_BLK, O), lambda i: (i, 0)),
            pl.BlockSpec((_BLK, O), lambda i: (i, 0)),
            pl.BlockSpec((_BLK, O), lambda i: (i, 0)),
        ],
        out_specs=pl.BlockSpec((_BLK, O), lambda i: (i, 0)),
        out_shape=jax.ShapeDtypeStruct((N_PAD, O), jnp.float32),
    )(a, s_col, p0, p1, t)


def _final_body(t_ref, r_ref, o_ref):
    y = t_ref[...] * r_ref[...]
    m = jnp.max(y, axis=1, keepdims=True)
    lse = jnp.log(jnp.sum(jnp.exp(y - m), axis=1, keepdims=True))
    o_ref[...] = y - m - lse


@jax.jit
def _final(t, rdinv_col):
    return pl.pallas_call(
        _final_body,
        grid=(_GRID,),
        in_specs=[
            pl.BlockSpec((_BLK, O), lambda i: (i, 0)),
            pl.BlockSpec((_BLK, 1), lambda i: (i, 0)),
        ],
        out_specs=pl.BlockSpec((_BLK, O), lambda i: (i, 0)),
        out_shape=jax.ShapeDtypeStruct((N_PAD, O), jnp.float32),
    )(t, rdinv_col)


def kernel(x, edge_index, W1, b1, W2, b2):
    src = edge_index[0].astype(jnp.int32)
    dst = edge_index[1].astype(jnp.int32)
    pad = jnp.full((E_PAD - E,), N_PAD - 1, dtype=jnp.int32)
    src_p = jnp.concatenate([src, pad])
    dst_p = jnp.concatenate([dst, pad])

    x_pad = jnp.pad(x, ((0, N_PAD - N), (0, 0)))
    h = _mlp(x_pad, W1, b1.reshape(1, H), W2, b2.reshape(1, O))

    zeros = jnp.zeros((N_PAD, O), jnp.float32)
    ones = jnp.ones((N_PAD, O), jnp.float32)

    pdeg = _sc_propagate(ones, src_p, dst_p, zeros)
    deg = pdeg[0, :, 0] + pdeg[1, :, 0] + 1.0
    valid = jnp.arange(N_PAD) < N
    dinv = jnp.where(valid, lax.rsqrt(deg), 0.0)
    rdinv = jnp.where(valid, jnp.sqrt(deg), 0.0)

    t = dinv[:, None] * h
    a = ALPHA * t
    s_col = ((1.0 - ALPHA) * dinv * dinv)[:, None]

    for _ in range(K):
        p = _sc_propagate(t, src_p, dst_p, zeros)
        t = _update(a, s_col, p[0], p[1], t)

    res = _final(t, rdinv[:, None])
    return res[:N]


# speedup vs baseline: 8.5799x; 8.5799x over previous
import jax
import jax.numpy as jnp
from jax import lax
from jax.experimental import pallas as pl
from jax.experimental.pallas import tpu as pltpu
from jax.experimental.pallas import tpu_sc as plsc

N = 10000
E = 320000
D = 128
H = 64
O = 64
K = 10
ALPHA = 0.1

NC = 2
NS = 16
NW = NC * NS
CHUNK = 128
NCH = 79
E_PAD = NW * NCH * CHUNK
N_PAD = 10240
ROWS_PER_TILE = N_PAD // NS


def _sc_prop_body(t_hbm, src_hbm, dst_hbm, zeros_hbm, out_hbm,
                  idx_s, idx_d, rows, agg):
    cid = lax.axis_index("c")
    sid = lax.axis_index("s")
    wid = cid * NS + sid

    pltpu.sync_copy(zeros_hbm.at[pl.ds(sid * ROWS_PER_TILE, ROWS_PER_TILE)],
                    agg.at[pl.ds(sid * ROWS_PER_TILE, ROWS_PER_TILE)])
    plsc.subcore_barrier()

    @pl.loop(0, NCH)
    def _(j):
        off = (wid * NCH + j) * CHUNK
        pltpu.sync_copy(src_hbm.at[pl.ds(off, CHUNK)], idx_s)
        pltpu.sync_copy(dst_hbm.at[pl.ds(off, CHUNK)], idx_d)
        pltpu.sync_copy(t_hbm.at[idx_s], rows)
        pltpu.sync_copy(rows, agg.at[idx_d], add=True)

    plsc.subcore_barrier()
    pltpu.sync_copy(agg.at[pl.ds(sid * ROWS_PER_TILE, ROWS_PER_TILE)],
                    out_hbm.at[cid].at[pl.ds(sid * ROWS_PER_TILE, ROWS_PER_TILE)])


@jax.jit
def _sc_propagate(t, src, dst, zeros):
    mesh = plsc.VectorSubcoreMesh(core_axis_name="c", subcore_axis_name="s")
    kern = pl.kernel(
        _sc_prop_body,
        out_type=jax.ShapeDtypeStruct((NC, N_PAD, O), jnp.float32),
        mesh=mesh,
        compiler_params=pltpu.CompilerParams(use_tc_tiling_on_sc=False),
        scratch_types=[
            pltpu.VMEM((CHUNK,), jnp.int32),
            pltpu.VMEM((CHUNK,), jnp.int32),
            pltpu.VMEM((CHUNK, O), jnp.float32),
            pltpu.VMEM_SHARED((N_PAD, O), jnp.float32),
        ],
    )
    return kern(t, src, dst, zeros)


_BLK = 1024
_GRID = N_PAD // _BLK


def _mlp_body(x_ref, w1_ref, b1_ref, w2_ref, b2_ref, h_ref):
    z = jnp.dot(x_ref[...], w1_ref[...], preferred_element_type=jnp.float32)
    z = jnp.maximum(z + b1_ref[...], 0.0)
    h_ref[...] = (jnp.dot(z, w2_ref[...], preferred_element_type=jnp.float32)
                  + b2_ref[...])


@jax.jit
def _mlp(x_pad, W1, b1, W2, b2):
    return pl.pallas_call(
        _mlp_body,
        grid=(_GRID,),
        in_specs=[
            pl.BlockSpec((_BLK, D), lambda i: (i, 0)),
            pl.BlockSpec((D, H), lambda i: (0, 0)),
            pl.BlockSpec((1, H), lambda i: (0, 0)),
            pl.BlockSpec((H, O), lambda i: (0, 0)),
            pl.BlockSpec((1, O), lambda i: (0, 0)),
        ],
        out_specs=pl.BlockSpec((_BLK, O), lambda i: (i, 0)),
        out_shape=jax.ShapeDtypeStruct((N_PAD, O), jnp.float32),
    )(x_pad, W1, b1, W2, b2)


def _update_body(a_ref, s_ref, p0_ref, p1_ref, t_ref, o_ref):
    o_ref[...] = a_ref[...] + s_ref[...] * (p0_ref[...] + p1_ref[...] + t_ref[...])


@jax.jit
def _update(a, s_col, p0, p1, t):
    return pl.pallas_call(
        _update_body,
        grid=(_GRID,),
        in_specs=[
            pl.BlockSpec((_BLK, O), lambda i: (i, 0)),
            pl.BlockSpec((_BLK, 1), lambda i: (i, 0)),
            pl.BlockSpec((_BLK, O), lambda i: (i, 0)),
            pl.BlockSpec((_BLK, O), lambda i: (i, 0)),
            pl.BlockSpec((_BLK, O), lambda i: (i, 0)),
        ],
        out_specs=pl.BlockSpec((_BLK, O), lambda i: (i, 0)),
        out_shape=jax.ShapeDtypeStruct((N_PAD, O), jnp.float32),
    )(a, s_col, p0, p1, t)


def _final_body(t_ref, r_ref, o_ref):
    y = t_ref[...] * r_ref[...]
    m = jnp.max(y, axis=1, keepdims=True)
    lse = jnp.log(jnp.sum(jnp.exp(y - m), axis=1, keepdims=True))
    o_ref[...] = y - m - lse


@jax.jit
def _final(t, rdinv_col):
    return pl.pallas_call(
        _final_body,
        grid=(_GRID,),
        in_specs=[
            pl.BlockSpec((_BLK, O), lambda i: (i, 0)),
            pl.BlockSpec((_BLK, 1), lambda i: (i, 0)),
        ],
        out_specs=pl.BlockSpec((_BLK, O), lambda i: (i, 0)),
        out_shape=jax.ShapeDtypeStruct((N_PAD, O), jnp.float32),
    )(t, rdinv_col)


def kernel(x, edge_index, W1, b1, W2, b2):
    src = edge_index[0].astype(jnp.int32)
    dst = edge_index[1].astype(jnp.int32)
    pad = jnp.full((E_PAD - E,), N_PAD - 1, dtype=jnp.int32)
    src_p = jnp.concatenate([src, pad])
    dst_p = jnp.concatenate([dst, pad])

    x_pad = jnp.pad(x, ((0, N_PAD - N), (0, 0)))
    h = _mlp(x_pad, W1, b1.reshape(1, H), W2, b2.reshape(1, O))

    zeros = jnp.zeros((N_PAD, O), jnp.float32)
    ones = jnp.ones((N_PAD, O), jnp.float32)

    pdeg = _sc_propagate(ones, src_p, dst_p, zeros)
    deg = pdeg[0, :, 0] + pdeg[1, :, 0] + 1.0
    valid = jnp.arange(N_PAD) < N
    dinv = jnp.where(valid, lax.rsqrt(deg), 0.0)
    rdinv = jnp.where(valid, jnp.sqrt(deg), 0.0)

    t = dinv[:, None] * h
    a = ALPHA * t
    s_col = ((1.0 - ALPHA) * dinv * dinv)[:, None]

    for _ in range(K):
        p = _sc_propagate(t, src_p, dst_p, zeros)
        t = _update(a, s_col, p[0], p[1], t)

    res = _final(t, rdinv[:, None])
    return res[:N]


# speedup vs baseline: 9.5945x; 1.1182x over previous
import jax
import jax.numpy as jnp
from jax import lax
from jax.experimental import pallas as pl
from jax.experimental.pallas import tpu as pltpu
from jax.experimental.pallas import tpu_sc as plsc

N = 10000
E = 320000
D = 128
H = 64
O = 64
K = 10
ALPHA = 0.1

NC = 2
NS = 16
NW = NC * NS
CHUNK = 128
NCH = 80
G = 5
NG = NCH // G
E_PAD = NW * NCH * CHUNK
ECH = E_PAD // CHUNK
N_PAD = 10240
ROWS_PER_TILE = N_PAD // NS
DEG_W = 16
DEG_SUP = 8
_SC_PARAMS = pltpu.CompilerParams(use_tc_tiling_on_sc=False)


def _sc_prop_body(t_hbm, src_hbm, dst_hbm, zeros_hbm, out_hbm,
                  idx_s, idx_d, rows, agg,
                  sem_i0, sem_i1, sem_g0, sem_g1, sem_s0, sem_s1):
    sems_i = (sem_i0, sem_i1)
    sems_g = (sem_g0, sem_g1)
    sems_s = (sem_s0, sem_s1)
    cid = lax.axis_index("c")
    sid = lax.axis_index("s")
    wid = cid * NS + sid
    base_row = wid * NCH

    pltpu.sync_copy(zeros_hbm.at[pl.ds(sid * ROWS_PER_TILE, ROWS_PER_TILE)],
                    agg.at[pl.ds(sid * ROWS_PER_TILE, ROWS_PER_TILE)])
    plsc.subcore_barrier()

    def issue_idx(gg, q, s):
        pltpu.async_copy(src_hbm.at[pl.ds(base_row + gg * G, G)],
                         idx_s.at[pl.ds(q * G, G)], sems_i[s])
        pltpu.async_copy(dst_hbm.at[pl.ds(base_row + gg * G, G)],
                         idx_d.at[pl.ds(q * G, G)], sems_i[s])

    def drain_idx(gg, q, s):
        pltpu.make_async_copy(src_hbm.at[pl.ds(base_row + gg * G, G)],
                              idx_s.at[pl.ds(q * G, G)], sems_i[s]).wait()
        pltpu.make_async_copy(dst_hbm.at[pl.ds(base_row + gg * G, G)],
                              idx_d.at[pl.ds(q * G, G)], sems_i[s]).wait()

    def drain_scatters(s, q):
        for i in range(G):
            pltpu.make_async_copy(rows.at[s, pl.ds(i * CHUNK, CHUNK)],
                                  agg.at[idx_d.at[q * G + i]], sems_s[s]).wait()

    issue_idx(0, 0, 0)
    issue_idx(1, 1, 1)

    @pl.loop(0, NG, step=4)
    def _(g):
        for kk in range(4):
            q = kk
            s = kk % 2
            gg = g + kk
            @pl.when(gg >= 2)
            def _():
                drain_scatters(s, q)
            drain_idx(gg, q, s)
            for i in range(G):
                pltpu.async_copy(t_hbm.at[idx_s.at[q * G + i]],
                                 rows.at[s, pl.ds(i * CHUNK, CHUNK)], sems_g[s])
            @pl.when(gg + 2 < NG)
            def _():
                issue_idx(gg + 2, (q + 2) % 4, s)
            for i in range(G):
                pltpu.make_async_copy(t_hbm.at[idx_s.at[q * G + i]],
                                      rows.at[s, pl.ds(i * CHUNK, CHUNK)],
                                      sems_g[s]).wait()
            for i in range(G):
                pltpu.async_copy(rows.at[s, pl.ds(i * CHUNK, CHUNK)],
                                 agg.at[idx_d.at[q * G + i]], sems_s[s],
                                 add=True)

    drain_scatters(0, 2)
    drain_scatters(1, 3)

    plsc.subcore_barrier()
    pltpu.sync_copy(agg.at[pl.ds(sid * ROWS_PER_TILE, ROWS_PER_TILE)],
                    out_hbm.at[cid].at[pl.ds(sid * ROWS_PER_TILE, ROWS_PER_TILE)])


@jax.jit
def _sc_propagate(t, src2d, dst2d, zeros):
    mesh = plsc.VectorSubcoreMesh(core_axis_name="c", subcore_axis_name="s")
    kern = pl.kernel(
        _sc_prop_body,
        out_type=jax.ShapeDtypeStruct((NC, N_PAD, O), jnp.float32),
        mesh=mesh,
        compiler_params=_SC_PARAMS,
        scratch_types=[
            pltpu.VMEM((4 * G, CHUNK), jnp.int32),
            pltpu.VMEM((4 * G, CHUNK), jnp.int32),
            pltpu.VMEM((2, G * CHUNK, O), jnp.float32),
            pltpu.VMEM_SHARED((N_PAD, O), jnp.float32),
            pltpu.SemaphoreType.DMA,
            pltpu.SemaphoreType.DMA,
            pltpu.SemaphoreType.DMA,
            pltpu.SemaphoreType.DMA,
            pltpu.SemaphoreType.DMA,
            pltpu.SemaphoreType.DMA,
        ],
    )
    return kern(t, src2d, dst2d, zeros)


def _sc_deg_body(dst_hbm, ones_hbm, zeros_hbm, out_hbm,
                 idx_d, ones_v, agg, sem):
    cid = lax.axis_index("c")
    sid = lax.axis_index("s")
    wid = cid * NS + sid
    base_row = wid * NCH
    stripe = N_PAD // NS

    pltpu.sync_copy(ones_hbm, ones_v)
    pltpu.sync_copy(zeros_hbm.at[pl.ds(sid * stripe, stripe)],
                    agg.at[pl.ds(sid * stripe, stripe)])
    plsc.subcore_barrier()

    @pl.loop(0, NCH // DEG_SUP)
    def _(k):
        pltpu.sync_copy(dst_hbm.at[pl.ds(base_row + k * DEG_SUP, DEG_SUP)],
                        idx_d)
        for i in range(DEG_SUP):
            pltpu.async_copy(ones_v, agg.at[idx_d.at[i]], sem, add=True)
        for i in range(DEG_SUP):
            pltpu.make_async_copy(ones_v, agg.at[idx_d.at[i]], sem).wait()

    plsc.subcore_barrier()
    pltpu.sync_copy(agg.at[pl.ds(sid * stripe, stripe)],
                    out_hbm.at[cid].at[pl.ds(sid * stripe, stripe)])


@jax.jit
def _sc_degrees(dst2d, ones16, zeros16):
    mesh = plsc.VectorSubcoreMesh(core_axis_name="c", subcore_axis_name="s")
    kern = pl.kernel(
        _sc_deg_body,
        out_type=jax.ShapeDtypeStruct((NC, N_PAD, DEG_W), jnp.float32),
        mesh=mesh,
        compiler_params=_SC_PARAMS,
        scratch_types=[
            pltpu.VMEM((DEG_SUP, CHUNK), jnp.int32),
            pltpu.VMEM((CHUNK, DEG_W), jnp.float32),
            pltpu.VMEM_SHARED((N_PAD, DEG_W), jnp.float32),
            pltpu.SemaphoreType.DMA,
        ],
    )
    return kern(dst2d, ones16, zeros16)


_BLK = 1024
_GRID = N_PAD // _BLK


def _mlp_body(x_ref, w1_ref, b1_ref, w2_ref, b2_ref, h_ref):
    z = jnp.dot(x_ref[...], w1_ref[...], preferred_element_type=jnp.float32)
    z = jnp.maximum(z + b1_ref[...], 0.0)
    h_ref[...] = (jnp.dot(z, w2_ref[...], preferred_element_type=jnp.float32)
                  + b2_ref[...])


@jax.jit
def _mlp(x_pad, W1, b1, W2, b2):
    return pl.pallas_call(
        _mlp_body,
        grid=(_GRID,),
        in_specs=[
            pl.BlockSpec((_BLK, D), lambda i: (i, 0)),
            pl.BlockSpec((D, H), lambda i: (0, 0)),
            pl.BlockSpec((1, H), lambda i: (0, 0)),
            pl.BlockSpec((H, O), lambda i: (0, 0)),
            pl.BlockSpec((1, O), lambda i: (0, 0)),
        ],
        out_specs=pl.BlockSpec((_BLK, O), lambda i: (i, 0)),
        out_shape=jax.ShapeDtypeStruct((N_PAD, O), jnp.float32),
    )(x_pad, W1, b1, W2, b2)


def _update_body(a_ref, s_ref, p0_ref, p1_ref, t_ref, o_ref):
    o_ref[...] = a_ref[...] + s_ref[...] * (p0_ref[...] + p1_ref[...] + t_ref[...])


@jax.jit
def _update(a, s_col, p0, p1, t):
    return pl.pallas_call(
        _update_body,
        grid=(_GRID,),
        in_specs=[
            pl.BlockSpec((_BLK, O), lambda i: (i, 0)),
            pl.BlockSpec((_BLK, 1), lambda i: (i, 0)),
            pl.BlockSpec((_BLK, O), lambda i: (i, 0)),
            pl.BlockSpec((_BLK, O), lambda i: (i, 0)),
            pl.BlockSpec((_BLK, O), lambda i: (i, 0)),
        ],
        out_specs=pl.BlockSpec((_BLK, O), lambda i: (i, 0)),
        out_shape=jax.ShapeDtypeStruct((N_PAD, O), jnp.float32),
    )(a, s_col, p0, p1, t)


def _final_body(t_ref, r_ref, o_ref):
    y = t_ref[...] * r_ref[...]
    m = jnp.max(y, axis=1, keepdims=True)
    lse = jnp.log(jnp.sum(jnp.exp(y - m), axis=1, keepdims=True))
    o_ref[...] = y - m - lse


@jax.jit
def _final(t, rdinv_col):
    return pl.pallas_call(
        _final_body,
        grid=(_GRID,),
        in_specs=[
            pl.BlockSpec((_BLK, O), lambda i: (i, 0)),
            pl.BlockSpec((_BLK, 1), lambda i: (i, 0)),
        ],
        out_specs=pl.BlockSpec((_BLK, O), lambda i: (i, 0)),
        out_shape=jax.ShapeDtypeStruct((N_PAD, O), jnp.float32),
    )(t, rdinv_col)


def kernel(x, edge_index, W1, b1, W2, b2):
    src = edge_index[0].astype(jnp.int32)
    dst = edge_index[1].astype(jnp.int32)
    npad = E_PAD - E
    pad_src = jnp.full((npad,), N_PAD - 1, dtype=jnp.int32)
    pad_dst = N + (jnp.arange(npad, dtype=jnp.int32) % (N_PAD - N))
    src2d = jnp.concatenate([src, pad_src]).reshape(ECH, CHUNK)
    dst2d = jnp.concatenate([dst, pad_dst]).reshape(ECH, CHUNK)

    x_pad = jnp.pad(x, ((0, N_PAD - N), (0, 0)))
    h = _mlp(x_pad, W1, b1.reshape(1, H), W2, b2.reshape(1, O))

    zeros = jnp.zeros((N_PAD, O), jnp.float32)
    ones16 = jnp.ones((CHUNK, DEG_W), jnp.float32)
    zeros16 = jnp.zeros((N_PAD, DEG_W), jnp.float32)

    pdeg = _sc_degrees(dst2d, ones16, zeros16)
    deg = pdeg[0, :, 0] + pdeg[1, :, 0] + 1.0
    valid = jnp.arange(N_PAD) < N
    dinv = jnp.where(valid, lax.rsqrt(deg), 0.0)
    rdinv = jnp.where(valid, jnp.sqrt(deg), 0.0)

    t = dinv[:, None] * h
    a = ALPHA * t
    s_col = ((1.0 - ALPHA) * dinv * dinv)[:, None]

    for _ in range(K):
        p = _sc_propagate(t, src2d, dst2d, zeros)
        t = _update(a, s_col, p[0], p[1], t)

    res = _final(t, rdinv[:, None])
    return res[:N]


# speedup vs baseline: 29.6529x; 3.0906x over previous
import jax
import jax.numpy as jnp
from jax import lax
from jax.experimental import pallas as pl
from jax.experimental.pallas import tpu as pltpu
from jax.experimental.pallas import tpu_sc as plsc

N = 10000
E = 320000
D = 128
H = 64
O = 64
K = 10
ALPHA = 0.1

NC = 2
NS = 16
HALF = O // NC
CHUNK = 128
NCH_T = 160
G = 5
NG = NCH_T // G
E_PAD = NS * NCH_T * CHUNK
ECH = E_PAD // CHUNK
N_PAD = 10240
RPT = N_PAD // NS
DEG_W = 16
DEG_SUP = 8
_SC_PARAMS = pltpu.CompilerParams(use_tc_tiling_on_sc=False)


def _sc_loop_body(t2_hbm, a2_hbm, sb_hbm, src_hbm, dst_hbm, out_hbm,
                  idx_s, idx_d, rows, tv, ar, agg, sh_t,
                  sem_i0, sem_i1, sem_g0, sem_g1, sem_s0, sem_s1):
    sems_i = (sem_i0, sem_i1)
    sems_g = (sem_g0, sem_g1)
    sems_s = (sem_s0, sem_s1)
    cid = lax.axis_index("c")
    sid = lax.axis_index("s")
    stripe = pl.ds(sid * RPT, RPT)
    base_row = sid * NCH_T

    def zero_agg_stripe():
        @pl.loop(0, RPT)
        def _(r):
            rows[0, r, pl.ds(0, 16)] = jnp.zeros((16,), jnp.float32)
            rows[0, r, pl.ds(16, 16)] = jnp.zeros((16,), jnp.float32)
        pltpu.sync_copy(rows.at[0], agg.at[stripe])

    pltpu.sync_copy(t2_hbm.at[cid].at[stripe], tv)
    pltpu.sync_copy(tv, sh_t.at[stripe])
    pltpu.sync_copy(a2_hbm.at[cid].at[stripe], ar)
    zero_agg_stripe()
    plsc.subcore_barrier()

    def issue_idx(gg, q, s):
        pltpu.async_copy(src_hbm.at[pl.ds(base_row + gg * G, G)],
                         idx_s.at[pl.ds(q * G, G)], sems_i[s])
        pltpu.async_copy(dst_hbm.at[pl.ds(base_row + gg * G, G)],
                         idx_d.at[pl.ds(q * G, G)], sems_i[s])

    def drain_idx(gg, q, s):
        pltpu.make_async_copy(src_hbm.at[pl.ds(base_row + gg * G, G)],
                              idx_s.at[pl.ds(q * G, G)], sems_i[s]).wait()
        pltpu.make_async_copy(dst_hbm.at[pl.ds(base_row + gg * G, G)],
                              idx_d.at[pl.ds(q * G, G)], sems_i[s]).wait()

    def drain_scatters(s, q):
        for i in range(G):
            pltpu.make_async_copy(rows.at[s, pl.ds(i * CHUNK, CHUNK)],
                                  agg.at[idx_d.at[q * G + i]], sems_s[s]).wait()

    @pl.loop(0, K)
    def _(k):
        issue_idx(0, 0, 0)
        issue_idx(1, 1, 1)

        @pl.loop(0, NG, step=4)
        def _(g):
            for kk in range(4):
                q = kk
                s = kk % 2
                gg = g + kk
                @pl.when(gg >= 2)
                def _():
                    drain_scatters(s, q)
                drain_idx(gg, q, s)
                for i in range(G):
                    pltpu.async_copy(sh_t.at[idx_s.at[q * G + i]],
                                     rows.at[s, pl.ds(i * CHUNK, CHUNK)],
                                     sems_g[s])
                @pl.when(gg + 2 < NG)
                def _():
                    issue_idx(gg + 2, (q + 2) % 4, s)
                for i in range(G):
                    pltpu.make_async_copy(sh_t.at[idx_s.at[q * G + i]],
                                          rows.at[s, pl.ds(i * CHUNK, CHUNK)],
                                          sems_g[s]).wait()
                for i in range(G):
                    pltpu.async_copy(rows.at[s, pl.ds(i * CHUNK, CHUNK)],
                                     agg.at[idx_d.at[q * G + i]], sems_s[s],
                                     add=True)

        drain_scatters(0, 2)
        drain_scatters(1, 3)
        plsc.subcore_barrier()

        pltpu.sync_copy(agg.at[stripe], rows.at[0])
        pltpu.sync_copy(sb_hbm.at[stripe], rows.at[1])

        @pl.loop(0, RPT)
        def _(r):
            for cp in (0, 16):
                cs = pl.ds(cp, 16)
                tv[r, cs] = ar[r, cs] + rows[1, r, cs] * (rows[0, r, cs]
                                                         + tv[r, cs])

        pltpu.sync_copy(tv, sh_t.at[stripe])
        zero_agg_stripe()
        plsc.subcore_barrier()

    pltpu.sync_copy(tv, out_hbm.at[cid].at[stripe])


@jax.jit
def _sc_k_loop(t2, a2, sb, src2d, dst2d):
    mesh = plsc.VectorSubcoreMesh(core_axis_name="c", subcore_axis_name="s")
    kern = pl.kernel(
        _sc_loop_body,
        out_type=jax.ShapeDtypeStruct((NC, N_PAD, HALF), jnp.float32),
        mesh=mesh,
        compiler_params=_SC_PARAMS,
        scratch_types=[
            pltpu.VMEM((4 * G, CHUNK), jnp.int32),
            pltpu.VMEM((4 * G, CHUNK), jnp.int32),
            pltpu.VMEM((2, G * CHUNK, HALF), jnp.float32),
            pltpu.VMEM((RPT, HALF), jnp.float32),
            pltpu.VMEM((RPT, HALF), jnp.float32),
            pltpu.VMEM_SHARED((N_PAD, HALF), jnp.float32),
            pltpu.VMEM_SHARED((N_PAD, HALF), jnp.float32),
            pltpu.SemaphoreType.DMA,
            pltpu.SemaphoreType.DMA,
            pltpu.SemaphoreType.DMA,
            pltpu.SemaphoreType.DMA,
            pltpu.SemaphoreType.DMA,
            pltpu.SemaphoreType.DMA,
        ],
    )
    return kern(t2, a2, sb, src2d, dst2d)


DEG_NCH = ECH // (NC * NS)


def _sc_deg_body(dst_hbm, ones_hbm, zeros_hbm, out_hbm,
                 idx_d, ones_v, agg, sem):
    cid = lax.axis_index("c")
    sid = lax.axis_index("s")
    wid = cid * NS + sid
    base_row = wid * DEG_NCH
    stripe = N_PAD // NS

    pltpu.sync_copy(ones_hbm, ones_v)
    pltpu.sync_copy(zeros_hbm.at[pl.ds(sid * stripe, stripe)],
                    agg.at[pl.ds(sid * stripe, stripe)])
    plsc.subcore_barrier()

    @pl.loop(0, DEG_NCH // DEG_SUP)
    def _(k):
        pltpu.sync_copy(dst_hbm.at[pl.ds(base_row + k * DEG_SUP, DEG_SUP)],
                        idx_d)
        for i in range(DEG_SUP):
            pltpu.async_copy(ones_v, agg.at[idx_d.at[i]], sem, add=True)
        for i in range(DEG_SUP):
            pltpu.make_async_copy(ones_v, agg.at[idx_d.at[i]], sem).wait()

    plsc.subcore_barrier()
    pltpu.sync_copy(agg.at[pl.ds(sid * stripe, stripe)],
                    out_hbm.at[cid].at[pl.ds(sid * stripe, stripe)])


@jax.jit
def _sc_degrees(dst2d, ones16, zeros16):
    mesh = plsc.VectorSubcoreMesh(core_axis_name="c", subcore_axis_name="s")
    kern = pl.kernel(
        _sc_deg_body,
        out_type=jax.ShapeDtypeStruct((NC, N_PAD, DEG_W), jnp.float32),
        mesh=mesh,
        compiler_params=_SC_PARAMS,
        scratch_types=[
            pltpu.VMEM((DEG_SUP, CHUNK), jnp.int32),
            pltpu.VMEM((CHUNK, DEG_W), jnp.float32),
            pltpu.VMEM_SHARED((N_PAD, DEG_W), jnp.float32),
            pltpu.SemaphoreType.DMA,
        ],
    )
    return kern(dst2d, ones16, zeros16)


_BLK = 1024
_GRID = N_PAD // _BLK


def _mlp_body(x_ref, w1_ref, b1_ref, w2_ref, b2_ref, h_ref):
    z = jnp.dot(x_ref[...], w1_ref[...], preferred_element_type=jnp.float32)
    z = jnp.maximum(z + b1_ref[...], 0.0)
    h_ref[...] = (jnp.dot(z, w2_ref[...], preferred_element_type=jnp.float32)
                  + b2_ref[...])


@jax.jit
def _mlp(x_pad, W1, b1, W2, b2):
    return pl.pallas_call(
        _mlp_body,
        grid=(_GRID,),
        in_specs=[
            pl.BlockSpec((_BLK, D), lambda i: (i, 0)),
            pl.BlockSpec((D, H), lambda i: (0, 0)),
            pl.BlockSpec((1, H), lambda i: (0, 0)),
            pl.BlockSpec((H, O), lambda i: (0, 0)),
            pl.BlockSpec((1, O), lambda i: (0, 0)),
        ],
        out_specs=pl.BlockSpec((_BLK, O), lambda i: (i, 0)),
        out_shape=jax.ShapeDtypeStruct((N_PAD, O), jnp.float32),
    )(x_pad, W1, b1, W2, b2)


def _final_body(t_ref, r_ref, o_ref):
    y = t_ref[...] * r_ref[...]
    m = jnp.max(y, axis=1, keepdims=True)
    lse = jnp.log(jnp.sum(jnp.exp(y - m), axis=1, keepdims=True))
    o_ref[...] = y - m - lse


@jax.jit
def _final(t, rdinv_col):
    return pl.pallas_call(
        _final_body,
        grid=(_GRID,),
        in_specs=[
            pl.BlockSpec((_BLK, O), lambda i: (i, 0)),
            pl.BlockSpec((_BLK, 1), lambda i: (i, 0)),
        ],
        out_specs=pl.BlockSpec((_BLK, O), lambda i: (i, 0)),
        out_shape=jax.ShapeDtypeStruct((N_PAD, O), jnp.float32),
    )(t, rdinv_col)


def kernel(x, edge_index, W1, b1, W2, b2):
    src = edge_index[0].astype(jnp.int32)
    dst = edge_index[1].astype(jnp.int32)
    npad = E_PAD - E
    pad_src = jnp.full((npad,), N_PAD - 1, dtype=jnp.int32)
    pad_dst = N + (jnp.arange(npad, dtype=jnp.int32) % (N_PAD - N))
    src2d = jnp.concatenate([src, pad_src]).reshape(ECH, CHUNK)
    dst2d = jnp.concatenate([dst, pad_dst]).reshape(ECH, CHUNK)

    x_pad = jnp.pad(x, ((0, N_PAD - N), (0, 0)))
    h = _mlp(x_pad, W1, b1.reshape(1, H), W2, b2.reshape(1, O))

    ones16 = jnp.ones((CHUNK, DEG_W), jnp.float32)
    zeros16 = jnp.zeros((N_PAD, DEG_W), jnp.float32)

    pdeg = _sc_degrees(dst2d, ones16, zeros16)
    deg = pdeg[0, :, 0] + pdeg[1, :, 0] + 1.0
    valid = jnp.arange(N_PAD) < N
    dinv = jnp.where(valid, lax.rsqrt(deg), 0.0)
    rdinv = jnp.where(valid, jnp.sqrt(deg), 0.0)

    t = dinv[:, None] * h
    a = ALPHA * t
    s = (1.0 - ALPHA) * dinv * dinv
    sb = jnp.broadcast_to(s[:, None], (N_PAD, HALF))
    t2 = jnp.stack([t[:, :HALF], t[:, HALF:]])
    a2 = jnp.stack([a[:, :HALF], a[:, HALF:]])

    tk2 = _sc_k_loop(t2, a2, sb, src2d, dst2d)
    tk = jnp.concatenate([tk2[0], tk2[1]], axis=1)

    res = _final(tk, rdinv[:, None])
    return res[:N]


# speedup vs baseline: 31.5461x; 1.0638x over previous
import jax
import jax.numpy as jnp
from jax import lax
from jax.experimental import pallas as pl
from jax.experimental.pallas import tpu as pltpu
from jax.experimental.pallas import tpu_sc as plsc

N = 10000
E = 320000
D = 128
H = 64
O = 64
K = 10
ALPHA = 0.1

NC = 2
NS = 16
HALF = O // NC
CHUNK = 128
NCH_T = 160
G = 5
NG = NCH_T // G
E_PAD = NS * NCH_T * CHUNK
ECH = E_PAD // CHUNK
N_PAD = 10240
RPT = N_PAD // NS
DEG_W = 16
DEG_SUP = 8
_SC_PARAMS = pltpu.CompilerParams(use_tc_tiling_on_sc=False)


def _sc_loop_body(t2_hbm, sb_hbm, src_hbm, dst_hbm, out_hbm,
                  idx_s, idx_d, rows, tv, ar, agg, sh_t,
                  sem_i0, sem_i1, sem_g0, sem_g1, sem_s0, sem_s1):
    sems_i = (sem_i0, sem_i1)
    sems_g = (sem_g0, sem_g1)
    sems_s = (sem_s0, sem_s1)
    cid = lax.axis_index("c")
    sid = lax.axis_index("s")
    stripe = pl.ds(sid * RPT, RPT)
    base_row = sid * NCH_T

    def zero_agg_stripe():
        @pl.loop(0, RPT)
        def _(r):
            rows[0, r, pl.ds(0, 16)] = jnp.zeros((16,), jnp.float32)
            rows[0, r, pl.ds(16, 16)] = jnp.zeros((16,), jnp.float32)
        pltpu.sync_copy(rows.at[0], agg.at[stripe])

    pltpu.sync_copy(t2_hbm.at[cid].at[stripe], tv)
    pltpu.sync_copy(tv, sh_t.at[stripe])

    @pl.loop(0, RPT)
    def _(r):
        for cp in (0, 16):
            cs = pl.ds(cp, 16)
            ar[r, cs] = ALPHA * tv[r, cs]

    zero_agg_stripe()
    plsc.subcore_barrier()

    def issue_idx(gg, q, s):
        pltpu.async_copy(src_hbm.at[pl.ds(base_row + gg * G, G)],
                         idx_s.at[pl.ds(q * G, G)], sems_i[s])
        pltpu.async_copy(dst_hbm.at[pl.ds(base_row + gg * G, G)],
                         idx_d.at[pl.ds(q * G, G)], sems_i[s])

    def drain_idx(gg, q, s):
        pltpu.make_async_copy(src_hbm.at[pl.ds(base_row + gg * G, G)],
                              idx_s.at[pl.ds(q * G, G)], sems_i[s]).wait()
        pltpu.make_async_copy(dst_hbm.at[pl.ds(base_row + gg * G, G)],
                              idx_d.at[pl.ds(q * G, G)], sems_i[s]).wait()

    def drain_scatters(s, q):
        for i in range(G):
            pltpu.make_async_copy(rows.at[s, pl.ds(i * CHUNK, CHUNK)],
                                  agg.at[idx_d.at[q * G + i]], sems_s[s]).wait()

    issue_idx(0, 0, 0)
    issue_idx(1, 1, 1)

    @pl.loop(0, K)
    def _(k):
        @pl.loop(0, NG, step=4)
        def _(g):
            for kk in range(4):
                q = kk
                s = kk % 2
                gg = g + kk
                @pl.when(gg >= 2)
                def _():
                    drain_scatters(s, q)
                drain_idx(gg, q, s)
                for i in range(G):
                    pltpu.async_copy(sh_t.at[idx_s.at[q * G + i]],
                                     rows.at[s, pl.ds(i * CHUNK, CHUNK)],
                                     sems_g[s])
                @pl.when(gg + 2 < NG)
                def _():
                    issue_idx(gg + 2, (q + 2) % 4, s)
                for i in range(G):
                    pltpu.make_async_copy(sh_t.at[idx_s.at[q * G + i]],
                                          rows.at[s, pl.ds(i * CHUNK, CHUNK)],
                                          sems_g[s]).wait()
                    pltpu.async_copy(rows.at[s, pl.ds(i * CHUNK, CHUNK)],
                                     agg.at[idx_d.at[q * G + i]], sems_s[s],
                                     add=True)

        drain_scatters(0, 2)
        drain_scatters(1, 3)
        plsc.subcore_barrier()

        @pl.when(k < K - 1)
        def _():
            issue_idx(0, 0, 0)
            issue_idx(1, 1, 1)
        pltpu.async_copy(agg.at[stripe], rows.at[0], sem_g0)
        pltpu.async_copy(sb_hbm.at[stripe], rows.at[1], sem_g1)
        pltpu.make_async_copy(agg.at[stripe], rows.at[0], sem_g0).wait()
        pltpu.make_async_copy(sb_hbm.at[stripe], rows.at[1], sem_g1).wait()

        @pl.loop(0, RPT)
        def _(r):
            for cp in (0, 16):
                cs = pl.ds(cp, 16)
                tv[r, cs] = ar[r, cs] + rows[1, r, cs] * (rows[0, r, cs]
                                                         + tv[r, cs])

        pltpu.sync_copy(tv, sh_t.at[stripe])
        zero_agg_stripe()
        plsc.subcore_barrier()

    pltpu.sync_copy(tv, out_hbm.at[cid].at[stripe])


@jax.jit
def _sc_k_loop(t2, sb, src2d, dst2d):
    mesh = plsc.VectorSubcoreMesh(core_axis_name="c", subcore_axis_name="s")
    kern = pl.kernel(
        _sc_loop_body,
        out_type=jax.ShapeDtypeStruct((NC, N_PAD, HALF), jnp.float32),
        mesh=mesh,
        compiler_params=_SC_PARAMS,
        scratch_types=[
            pltpu.VMEM((4 * G, CHUNK), jnp.int32),
            pltpu.VMEM((4 * G, CHUNK), jnp.int32),
            pltpu.VMEM((2, G * CHUNK, HALF), jnp.float32),
            pltpu.VMEM((RPT, HALF), jnp.float32),
            pltpu.VMEM((RPT, HALF), jnp.float32),
            pltpu.VMEM_SHARED((N_PAD, HALF), jnp.float32),
            pltpu.VMEM_SHARED((N_PAD, HALF), jnp.float32),
            pltpu.SemaphoreType.DMA,
            pltpu.SemaphoreType.DMA,
            pltpu.SemaphoreType.DMA,
            pltpu.SemaphoreType.DMA,
            pltpu.SemaphoreType.DMA,
            pltpu.SemaphoreType.DMA,
        ],
    )
    return kern(t2, sb, src2d, dst2d)


DEG_NCH = ECH // (NC * NS)


def _sc_deg_body(dst_hbm, ones_hbm, zeros_hbm, out_hbm,
                 idx_d, ones_v, agg, sem):
    cid = lax.axis_index("c")
    sid = lax.axis_index("s")
    wid = cid * NS + sid
    base_row = wid * DEG_NCH
    stripe = N_PAD // NS

    pltpu.sync_copy(ones_hbm, ones_v)
    pltpu.sync_copy(zeros_hbm.at[pl.ds(sid * stripe, stripe)],
                    agg.at[pl.ds(sid * stripe, stripe)])
    plsc.subcore_barrier()

    @pl.loop(0, DEG_NCH // DEG_SUP)
    def _(k):
        pltpu.sync_copy(dst_hbm.at[pl.ds(base_row + k * DEG_SUP, DEG_SUP)],
                        idx_d)
        for i in range(DEG_SUP):
            pltpu.async_copy(ones_v, agg.at[idx_d.at[i]], sem, add=True)
        for i in range(DEG_SUP):
            pltpu.make_async_copy(ones_v, agg.at[idx_d.at[i]], sem).wait()

    plsc.subcore_barrier()
    pltpu.sync_copy(agg.at[pl.ds(sid * stripe, stripe)],
                    out_hbm.at[cid].at[pl.ds(sid * stripe, stripe)])


@jax.jit
def _sc_degrees(dst2d, ones16, zeros16):
    mesh = plsc.VectorSubcoreMesh(core_axis_name="c", subcore_axis_name="s")
    kern = pl.kernel(
        _sc_deg_body,
        out_type=jax.ShapeDtypeStruct((NC, N_PAD, DEG_W), jnp.float32),
        mesh=mesh,
        compiler_params=_SC_PARAMS,
        scratch_types=[
            pltpu.VMEM((DEG_SUP, CHUNK), jnp.int32),
            pltpu.VMEM((CHUNK, DEG_W), jnp.float32),
            pltpu.VMEM_SHARED((N_PAD, DEG_W), jnp.float32),
            pltpu.SemaphoreType.DMA,
        ],
    )
    return kern(dst2d, ones16, zeros16)


_BLK = 1024
_GRID = N_PAD // _BLK


def _mlp_body(x_ref, w1_ref, b1_ref, w2_ref, b2_ref, d_ref, t2_ref, sb_ref):
    z = jnp.dot(x_ref[...], w1_ref[...], preferred_element_type=jnp.float32)
    z = jnp.maximum(z + b1_ref[...], 0.0)
    hh = (jnp.dot(z, w2_ref[...], preferred_element_type=jnp.float32)
          + b2_ref[...])
    d = d_ref[...]
    t = hh * d
    t2_ref[...] = jnp.stack([t[:, :HALF], t[:, HALF:]])
    sb_ref[...] = jnp.broadcast_to((1.0 - ALPHA) * d * d, (_BLK, HALF))


@jax.jit
def _mlp(x_pad, W1, b1, W2, b2, dinv_col):
    return pl.pallas_call(
        _mlp_body,
        grid=(_GRID,),
        in_specs=[
            pl.BlockSpec((_BLK, D), lambda i: (i, 0)),
            pl.BlockSpec((D, H), lambda i: (0, 0)),
            pl.BlockSpec((1, H), lambda i: (0, 0)),
            pl.BlockSpec((H, O), lambda i: (0, 0)),
            pl.BlockSpec((1, O), lambda i: (0, 0)),
            pl.BlockSpec((_BLK, 1), lambda i: (i, 0)),
        ],
        out_specs=[
            pl.BlockSpec((NC, _BLK, HALF), lambda i: (0, i, 0)),
            pl.BlockSpec((_BLK, HALF), lambda i: (i, 0)),
        ],
        out_shape=[
            jax.ShapeDtypeStruct((NC, N_PAD, HALF), jnp.float32),
            jax.ShapeDtypeStruct((N_PAD, HALF), jnp.float32),
        ],
    )(x_pad, W1, b1, W2, b2, dinv_col)


_FBLK = 1000
_FGRID = N // _FBLK


def _final_body(tl_ref, tr_ref, r_ref, o_ref):
    y = jnp.concatenate([tl_ref[0], tr_ref[0]], axis=1) * r_ref[...]
    m = jnp.max(y, axis=1, keepdims=True)
    lse = jnp.log(jnp.sum(jnp.exp(y - m), axis=1, keepdims=True))
    o_ref[...] = y - m - lse


@jax.jit
def _final(tk2, rdinv_col):
    return pl.pallas_call(
        _final_body,
        grid=(_FGRID,),
        in_specs=[
            pl.BlockSpec((1, _FBLK, HALF), lambda i: (0, i, 0)),
            pl.BlockSpec((1, _FBLK, HALF), lambda i: (1, i, 0)),
            pl.BlockSpec((_FBLK, 1), lambda i: (i, 0)),
        ],
        out_specs=pl.BlockSpec((_FBLK, O), lambda i: (i, 0)),
        out_shape=jax.ShapeDtypeStruct((N, O), jnp.float32),
    )(tk2, tk2, rdinv_col)


def kernel(x, edge_index, W1, b1, W2, b2):
    src = edge_index[0].astype(jnp.int32)
    dst = edge_index[1].astype(jnp.int32)
    npad = E_PAD - E
    pad_src = jnp.full((npad,), N_PAD - 1, dtype=jnp.int32)
    pad_dst = N + (jnp.arange(npad, dtype=jnp.int32) % (N_PAD - N))
    src2d = jnp.concatenate([src, pad_src]).reshape(ECH, CHUNK)
    dst2d = jnp.concatenate([dst, pad_dst]).reshape(ECH, CHUNK)

    x_pad = jnp.pad(x, ((0, N_PAD - N), (0, 0)))

    ones16 = jnp.ones((CHUNK, DEG_W), jnp.float32)
    zeros16 = jnp.zeros((N_PAD, DEG_W), jnp.float32)

    pdeg = _sc_degrees(dst2d, ones16, zeros16)
    deg = pdeg[0, :, 0] + pdeg[1, :, 0] + 1.0
    valid = jnp.arange(N_PAD) < N
    dinv = jnp.where(valid, lax.rsqrt(deg), 0.0)
    rdinv = jnp.where(valid, jnp.sqrt(deg), 0.0)

    t2, sb = _mlp(x_pad, W1, b1.reshape(1, H), W2, b2.reshape(1, O),
                  dinv[:, None])
    tk2 = _sc_k_loop(t2, sb, src2d, dst2d)
    return _final(tk2, rdinv[:, None])


# speedup vs baseline: 32.9594x; 1.0448x over previous
import jax
import jax.numpy as jnp
from jax import lax
from jax.experimental import pallas as pl
from jax.experimental.pallas import tpu as pltpu
from jax.experimental.pallas import tpu_sc as plsc

N = 10000
E = 320000
D = 128
H = 64
O = 64
K = 10
ALPHA = 0.1

NC = 2
NS = 16
HALF = O // NC
CHUNK = 128
NCH_T = 160
G = 5
NG = NCH_T // G
E_PAD = NS * NCH_T * CHUNK
ECH = E_PAD // CHUNK
N_PAD = 10240
RPT = N_PAD // NS
DEG_W = 16
DEG_SUP = 8
_SC_PARAMS = pltpu.CompilerParams(use_tc_tiling_on_sc=False)


def _sc_loop_body(t2_hbm, sb_hbm, src_hbm, dst_hbm, out_hbm,
                  idx_s, idx_d, rows, tv, ar, agg, sh_t,
                  sem_i0, sem_i1, sem_g0, sem_g1, sem_s0, sem_s1):
    sems_i = (sem_i0, sem_i1)
    sems_g = (sem_g0, sem_g1)
    sems_s = (sem_s0, sem_s1)
    cid = lax.axis_index("c")
    sid = lax.axis_index("s")
    stripe = pl.ds(sid * RPT, RPT)
    base_row = sid * NCH_T

    def zero_agg_stripe():
        @pl.loop(0, RPT, step=4)
        def _(r):
            for rr in range(4):
                for cp in (0, 16):
                    rows[0, r + rr, pl.ds(cp, 16)] = jnp.zeros((16,),
                                                               jnp.float32)
        pltpu.sync_copy(rows.at[0], agg.at[stripe])

    pltpu.sync_copy(t2_hbm.at[cid].at[stripe], tv)
    pltpu.sync_copy(tv, sh_t.at[stripe])

    @pl.loop(0, RPT)
    def _(r):
        for cp in (0, 16):
            cs = pl.ds(cp, 16)
            ar[r, cs] = ALPHA * tv[r, cs]

    zero_agg_stripe()
    plsc.subcore_barrier()

    def issue_idx(gg, q, s):
        pltpu.async_copy(src_hbm.at[pl.ds(base_row + gg * G, G)],
                         idx_s.at[pl.ds(q * G, G)], sems_i[s])
        pltpu.async_copy(dst_hbm.at[pl.ds(base_row + gg * G, G)],
                         idx_d.at[pl.ds(q * G, G)], sems_i[s])

    def drain_idx(gg, q, s):
        pltpu.make_async_copy(src_hbm.at[pl.ds(base_row + gg * G, G)],
                              idx_s.at[pl.ds(q * G, G)], sems_i[s]).wait()
        pltpu.make_async_copy(dst_hbm.at[pl.ds(base_row + gg * G, G)],
                              idx_d.at[pl.ds(q * G, G)], sems_i[s]).wait()

    def drain_scatters(s, q):
        for i in range(G):
            pltpu.make_async_copy(rows.at[s, pl.ds(i * CHUNK, CHUNK)],
                                  agg.at[idx_d.at[q * G + i]], sems_s[s]).wait()

    issue_idx(0, 0, 0)
    issue_idx(1, 1, 1)

    @pl.loop(0, K)
    def _(k):
        @pl.loop(0, NG, step=4)
        def _(g):
            for kk in range(4):
                q = kk
                s = kk % 2
                gg = g + kk
                @pl.when(gg >= 2)
                def _():
                    drain_scatters(s, q)
                drain_idx(gg, q, s)
                for i in range(G):
                    pltpu.async_copy(sh_t.at[idx_s.at[q * G + i]],
                                     rows.at[s, pl.ds(i * CHUNK, CHUNK)],
                                     sems_g[s])
                @pl.when(gg + 2 < NG)
                def _():
                    issue_idx(gg + 2, (q + 2) % 4, s)
                for i in range(G):
                    pltpu.make_async_copy(sh_t.at[idx_s.at[q * G + i]],
                                          rows.at[s, pl.ds(i * CHUNK, CHUNK)],
                                          sems_g[s]).wait()
                    pltpu.async_copy(rows.at[s, pl.ds(i * CHUNK, CHUNK)],
                                     agg.at[idx_d.at[q * G + i]], sems_s[s],
                                     add=True)

        drain_scatters(0, 2)
        drain_scatters(1, 3)
        plsc.subcore_barrier()

        @pl.when(k < K - 1)
        def _():
            issue_idx(0, 0, 0)
            issue_idx(1, 1, 1)
        pltpu.async_copy(agg.at[stripe], rows.at[0], sem_g0)
        pltpu.async_copy(sb_hbm.at[stripe], rows.at[1], sem_g1)
        pltpu.make_async_copy(agg.at[stripe], rows.at[0], sem_g0).wait()
        pltpu.make_async_copy(sb_hbm.at[stripe], rows.at[1], sem_g1).wait()

        @pl.loop(0, RPT, step=4)
        def _(r):
            for rr in range(4):
                for cp in (0, 16):
                    cs = pl.ds(cp, 16)
                    tv[r + rr, cs] = (ar[r + rr, cs]
                                      + rows[1, r + rr, cs]
                                      * (rows[0, r + rr, cs] + tv[r + rr, cs]))

        pltpu.sync_copy(tv, sh_t.at[stripe])
        zero_agg_stripe()
        plsc.subcore_barrier()

    pltpu.sync_copy(tv, out_hbm.at[cid].at[stripe])


@jax.jit
def _sc_k_loop(t2, sb, src2d, dst2d):
    mesh = plsc.VectorSubcoreMesh(core_axis_name="c", subcore_axis_name="s")
    kern = pl.kernel(
        _sc_loop_body,
        out_type=jax.ShapeDtypeStruct((NC, N_PAD, HALF), jnp.float32),
        mesh=mesh,
        compiler_params=_SC_PARAMS,
        scratch_types=[
            pltpu.VMEM((4 * G, CHUNK), jnp.int32),
            pltpu.VMEM((4 * G, CHUNK), jnp.int32),
            pltpu.VMEM((2, G * CHUNK, HALF), jnp.float32),
            pltpu.VMEM((RPT, HALF), jnp.float32),
            pltpu.VMEM((RPT, HALF), jnp.float32),
            pltpu.VMEM_SHARED((N_PAD, HALF), jnp.float32),
            pltpu.VMEM_SHARED((N_PAD, HALF), jnp.float32),
            pltpu.SemaphoreType.DMA,
            pltpu.SemaphoreType.DMA,
            pltpu.SemaphoreType.DMA,
            pltpu.SemaphoreType.DMA,
            pltpu.SemaphoreType.DMA,
            pltpu.SemaphoreType.DMA,
        ],
    )
    return kern(t2, sb, src2d, dst2d)


DEG_NCH = ECH // (NC * NS)


def _sc_deg_body(dst_hbm, ones_hbm, zeros_hbm, out_hbm,
                 idx_d, ones_v, agg, sem):
    cid = lax.axis_index("c")
    sid = lax.axis_index("s")
    wid = cid * NS + sid
    base_row = wid * DEG_NCH
    stripe = N_PAD // NS

    pltpu.sync_copy(ones_hbm, ones_v)
    pltpu.sync_copy(zeros_hbm.at[pl.ds(sid * stripe, stripe)],
                    agg.at[pl.ds(sid * stripe, stripe)])
    plsc.subcore_barrier()

    @pl.loop(0, DEG_NCH // DEG_SUP)
    def _(k):
        pltpu.sync_copy(dst_hbm.at[pl.ds(base_row + k * DEG_SUP, DEG_SUP)],
                        idx_d)
        for i in range(DEG_SUP):
            pltpu.async_copy(ones_v, agg.at[idx_d.at[i]], sem, add=True)
        for i in range(DEG_SUP):
            pltpu.make_async_copy(ones_v, agg.at[idx_d.at[i]], sem).wait()

    plsc.subcore_barrier()
    pltpu.sync_copy(agg.at[pl.ds(sid * stripe, stripe)],
                    out_hbm.at[cid].at[pl.ds(sid * stripe, stripe)])


@jax.jit
def _sc_degrees(dst2d, ones16, zeros16):
    mesh = plsc.VectorSubcoreMesh(core_axis_name="c", subcore_axis_name="s")
    kern = pl.kernel(
        _sc_deg_body,
        out_type=jax.ShapeDtypeStruct((NC, N_PAD, DEG_W), jnp.float32),
        mesh=mesh,
        compiler_params=_SC_PARAMS,
        scratch_types=[
            pltpu.VMEM((DEG_SUP, CHUNK), jnp.int32),
            pltpu.VMEM((CHUNK, DEG_W), jnp.float32),
            pltpu.VMEM_SHARED((N_PAD, DEG_W), jnp.float32),
            pltpu.SemaphoreType.DMA,
        ],
    )
    return kern(dst2d, ones16, zeros16)


_BLK = 1024
_GRID = N_PAD // _BLK


def _mlp_body(x_ref, w1_ref, b1_ref, w2_ref, b2_ref, h_ref):
    z = jnp.dot(x_ref[...], w1_ref[...], preferred_element_type=jnp.float32)
    z = jnp.maximum(z + b1_ref[...], 0.0)
    h_ref[...] = (jnp.dot(z, w2_ref[...], preferred_element_type=jnp.float32)
                  + b2_ref[...])


@jax.jit
def _mlp(x_pad, W1, b1, W2, b2):
    return pl.pallas_call(
        _mlp_body,
        grid=(_GRID,),
        in_specs=[
            pl.BlockSpec((_BLK, D), lambda i: (i, 0)),
            pl.BlockSpec((D, H), lambda i: (0, 0)),
            pl.BlockSpec((1, H), lambda i: (0, 0)),
            pl.BlockSpec((H, O), lambda i: (0, 0)),
            pl.BlockSpec((1, O), lambda i: (0, 0)),
        ],
        out_specs=pl.BlockSpec((_BLK, O), lambda i: (i, 0)),
        out_shape=jax.ShapeDtypeStruct((N_PAD, O), jnp.float32),
    )(x_pad, W1, b1, W2, b2)


def _prep_body(h_ref, d_ref, t2_ref, sb_ref):
    d = d_ref[...]
    t = h_ref[...] * d
    t2_ref[...] = jnp.stack([t[:, :HALF], t[:, HALF:]])
    sb_ref[...] = jnp.broadcast_to((1.0 - ALPHA) * d * d, (_BLK, HALF))


@jax.jit
def _prep(h, dinv_col):
    return pl.pallas_call(
        _prep_body,
        grid=(_GRID,),
        in_specs=[
            pl.BlockSpec((_BLK, O), lambda i: (i, 0)),
            pl.BlockSpec((_BLK, 1), lambda i: (i, 0)),
        ],
        out_specs=[
            pl.BlockSpec((NC, _BLK, HALF), lambda i: (0, i, 0)),
            pl.BlockSpec((_BLK, HALF), lambda i: (i, 0)),
        ],
        out_shape=[
            jax.ShapeDtypeStruct((NC, N_PAD, HALF), jnp.float32),
            jax.ShapeDtypeStruct((N_PAD, HALF), jnp.float32),
        ],
    )(h, dinv_col)


_FBLK = 1000
_FGRID = N // _FBLK


def _final_body(tl_ref, tr_ref, r_ref, o_ref):
    y = jnp.concatenate([tl_ref[0], tr_ref[0]], axis=1) * r_ref[...]
    m = jnp.max(y, axis=1, keepdims=True)
    lse = jnp.log(jnp.sum(jnp.exp(y - m), axis=1, keepdims=True))
    o_ref[...] = y - m - lse


@jax.jit
def _final(tk2, rdinv_col):
    return pl.pallas_call(
        _final_body,
        grid=(_FGRID,),
        in_specs=[
            pl.BlockSpec((1, _FBLK, HALF), lambda i: (0, i, 0)),
            pl.BlockSpec((1, _FBLK, HALF), lambda i: (1, i, 0)),
            pl.BlockSpec((_FBLK, 1), lambda i: (i, 0)),
        ],
        out_specs=pl.BlockSpec((_FBLK, O), lambda i: (i, 0)),
        out_shape=jax.ShapeDtypeStruct((N, O), jnp.float32),
    )(tk2, tk2, rdinv_col)


def kernel(x, edge_index, W1, b1, W2, b2):
    src = edge_index[0].astype(jnp.int32)
    dst = edge_index[1].astype(jnp.int32)
    npad = E_PAD - E
    pad_src = jnp.full((npad,), N_PAD - 1, dtype=jnp.int32)
    pad_dst = N + (jnp.arange(npad, dtype=jnp.int32) % (N_PAD - N))
    src2d = jnp.concatenate([src, pad_src]).reshape(ECH, CHUNK)
    dst2d = jnp.concatenate([dst, pad_dst]).reshape(ECH, CHUNK)

    x_pad = jnp.pad(x, ((0, N_PAD - N), (0, 0)))

    ones16 = jnp.ones((CHUNK, DEG_W), jnp.float32)
    zeros16 = jnp.zeros((N_PAD, DEG_W), jnp.float32)

    pdeg = _sc_degrees(dst2d, ones16, zeros16)
    deg = pdeg[0, :, 0] + pdeg[1, :, 0] + 1.0
    valid = jnp.arange(N_PAD) < N
    dinv = jnp.where(valid, lax.rsqrt(deg), 0.0)
    rdinv = jnp.where(valid, jnp.sqrt(deg), 0.0)

    h = _mlp(x_pad, W1, b1.reshape(1, H), W2, b2.reshape(1, O))
    t2, sb = _prep(h, dinv[:, None])
    tk2 = _sc_k_loop(t2, sb, src2d, dst2d)
    return _final(tk2, rdinv[:, None])
